# baseline (speedup 1.0000x reference)
import functools
import jax
import jax.numpy as jnp
from jax import lax
from jax.experimental import pallas as pl
from jax.experimental.pallas import tpu as pltpu
from jax.experimental.pallas import tpu_sc as plsc

EPS = 1e-5
NC, NS, L = 2, 16, 16
NW = NC * NS
CHUNK = 16


def _rsqrt_nr(t):
    i = lax.bitcast_convert_type(t, jnp.int32)
    i = jnp.int32(0x5F3759DF) - lax.shift_right_arithmetic(i, 1)
    y = lax.bitcast_convert_type(i, jnp.float32)
    half = t * 0.5
    for _ in range(3):
        y = y * (1.5 - half * y * y)
    return y


def kernel(x, pos_emb, ln_gamma, ln_beta):
    B, S, D = x.shape
    vecs = D // L
    x2 = x.reshape(B * S, D)
    s_per_w = S // NW
    n_chunks = s_per_w // CHUNK
    T = n_chunks * B
    inv_d = 1.0 / D
    mesh = plsc.VectorSubcoreMesh(core_axis_name="c", subcore_axis_name="s")

    @functools.partial(
        pl.kernel,
        mesh=mesh,
        compiler_params=pltpu.CompilerParams(needs_layout_passes=False),
        out_type=jax.ShapeDtypeStruct((B * S, D), jnp.float32),
        scratch_types=[
            pltpu.VMEM((2, CHUNK, D), jnp.float32),
            pltpu.VMEM((2, CHUNK, D), jnp.float32),
            pltpu.VMEM((CHUNK, D), jnp.float32),
            pltpu.VMEM((D,), jnp.float32),
            pltpu.VMEM((D,), jnp.float32),
            pltpu.VMEM((CHUNK, L), jnp.float32),
            pltpu.SemaphoreType.DMA,
            pltpu.SemaphoreType.DMA,
        ],
    )
    def k(x_hbm, pos_hbm, g_hbm, b_hbm, out_hbm, xv, ov, pv, gv, bv, red, si, so):
        wid = lax.axis_index("s") * NC + lax.axis_index("c")
        s_base = wid * s_per_w
        pltpu.sync_copy(g_hbm, gv)
        pltpu.sync_copy(b_hbm, bv)
        iota = lax.iota(jnp.int32, L)

        def task_row(u):
            b = lax.bitwise_and(u, B - 1)
            c = lax.shift_right_logical(u, 2)
            return b * S + s_base + c * CHUNK

        def start_in(u, slot):
            pltpu.make_async_copy(
                x_hbm.at[pl.ds(task_row(u), CHUNK)], xv.at[slot], si
            ).start()

        def wait_chunk(sem):
            pltpu.make_async_copy(
                x_hbm.at[pl.ds(0, CHUNK)], xv.at[0], sem
            ).wait()

        def lanesum(v, i):
            for k_ in (1, 2, 4, 8):
                red[i, :] = v
                v = v + plsc.load_gather(red.at[i], [lax.bitwise_xor(iota, k_)])
            return v

        def compute_chunk(slot):
            @plsc.parallel_loop(0, CHUNK, unroll=1)
            def row_body(i):
                p1 = [jnp.zeros((L,), jnp.float32) for _ in range(4)]
                p2 = [jnp.zeros((L,), jnp.float32) for _ in range(4)]
                for j in range(vecs):
                    sl = pl.ds(j * L, L)
                    e = xv[slot, i, sl] + pv[i, sl]
                    ov[slot, i, sl] = e
                    p1[j % 4] = p1[j % 4] + e
                    p2[j % 4] = p2[j % 4] + e * e
                a1 = (p1[0] + p1[1]) + (p1[2] + p1[3])
                a2 = (p2[0] + p2[1]) + (p2[2] + p2[3])
                mean = lanesum(a1, i) * inv_d
                msq = lanesum(a2, i) * inv_d
                inv = _rsqrt_nr(msq - mean * mean + EPS)
                ov[slot, i, pl.ds(0, L)] = inv + mean

        start_in(0, 0)
        start_in(1, 1)

        def task_body(t, _):
            slot = lax.bitwise_and(t, 1)

            @pl.when(lax.bitwise_and(t, B - 1) == 0)
            def _():
                srow = s_base + lax.shift_right_logical(t, 2) * CHUNK
                pltpu.sync_copy(pos_hbm.at[pl.ds(srow, CHUNK)], pv)

            wait_chunk(si)

            @pl.when(t >= 2)
            def _():
                wait_chunk(so)

            compute_chunk(slot)
            pltpu.make_async_copy(
                ov.at[slot], out_hbm.at[pl.ds(task_row(t), CHUNK)], so
            ).start()

            @pl.when(t + 2 < T)
            def _():
                start_in(t + 2, slot)

            return 0

        lax.fori_loop(0, T, task_body, 0)
        wait_chunk(so)
        wait_chunk(so)

    return k(x2, pos_emb, ln_gamma, ln_beta).reshape(B, S, D)


# speedup vs baseline: 1.5209x; 1.5209x over previous
import functools
import jax
import jax.numpy as jnp
from jax import lax
from jax.experimental import pallas as pl
from jax.experimental.pallas import tpu as pltpu
from jax.experimental.pallas import tpu_sc as plsc

EPS = 1e-5
NC, NS, L = 2, 16, 16
NW = NC * NS
CHUNK = 16


def _rsqrt_nr(t):
    i = lax.bitcast_convert_type(t, jnp.int32)
    i = jnp.int32(0x5F3759DF) - lax.shift_right_arithmetic(i, 1)
    y = lax.bitcast_convert_type(i, jnp.float32)
    half = t * 0.5
    for _ in range(3):
        y = y * (1.5 - half * y * y)
    return y


def kernel(x, pos_emb, ln_gamma, ln_beta):
    B, S, D = x.shape
    vecs = D // L
    x2 = x.reshape(B * S, D)
    s_per_w = S // NW
    n_chunks = s_per_w // CHUNK
    T = n_chunks * B
    inv_d = 1.0 / D
    mesh = plsc.VectorSubcoreMesh(core_axis_name="c", subcore_axis_name="s")

    @functools.partial(
        pl.kernel,
        mesh=mesh,
        compiler_params=pltpu.CompilerParams(needs_layout_passes=False),
        out_type=jax.ShapeDtypeStruct((B * S, D), jnp.float32),
        scratch_types=[
            pltpu.VMEM((2, CHUNK, D), jnp.float32),
            pltpu.VMEM((2, CHUNK, D), jnp.float32),
            pltpu.VMEM((CHUNK, D), jnp.float32),
            pltpu.VMEM((D,), jnp.float32),
            pltpu.VMEM((D,), jnp.float32),
            pltpu.VMEM((CHUNK, L), jnp.float32),
            pltpu.SemaphoreType.DMA,
            pltpu.SemaphoreType.DMA,
        ],
    )
    def k(x_hbm, pos_hbm, g_hbm, b_hbm, out_hbm, xv, ov, pv, gv, bv, red, si, so):
        wid = lax.axis_index("s") * NC + lax.axis_index("c")
        s_base = wid * s_per_w
        pltpu.sync_copy(g_hbm, gv)
        pltpu.sync_copy(b_hbm, bv)
        iota = lax.iota(jnp.int32, L)

        def task_row(u):
            b = lax.bitwise_and(u, B - 1)
            c = lax.shift_right_logical(u, 2)
            return b * S + s_base + c * CHUNK

        def start_in(u, slot):
            pltpu.make_async_copy(
                x_hbm.at[pl.ds(task_row(u), CHUNK)], xv.at[slot], si
            ).start()

        def wait_chunk(sem):
            pltpu.make_async_copy(
                x_hbm.at[pl.ds(0, CHUNK)], xv.at[0], sem
            ).wait()

        dnums = lax.GatherDimensionNumbers(
            offset_dims=(), collapsed_slice_dims=(0,), start_index_map=(0,))

        def lanesum(v):
            for k_ in (1, 2, 4, 8):
                perm = lax.gather(
                    v, lax.bitwise_xor(iota, k_)[:, None], dnums,
                    slice_sizes=(1,),
                    mode=lax.GatherScatterMode.PROMISE_IN_BOUNDS)
                v = v + perm
            return v

        def compute_chunk(slot):
            def row_body(i, _):
                zero = jnp.zeros((L,), jnp.float32)

                @plsc.parallel_loop(0, vecs, unroll=4, carry=(zero, zero))
                def acc(j, c):
                    a1, a2 = c
                    sl = pl.ds(j * L, L)
                    e = xv[slot, i, sl] + pv[i, sl]
                    ov[slot, i, sl] = e
                    return a1 + e, a2 + e * e

                a1, a2 = acc
                mean = lanesum(a1) * inv_d
                msq = lanesum(a2) * inv_d
                inv = _rsqrt_nr(msq - mean * mean + EPS)
                shift = mean * inv

                @plsc.parallel_loop(0, vecs, unroll=4)
                def norm(j):
                    sl = pl.ds(j * L, L)
                    e = ov[slot, i, sl]
                    ov[slot, i, sl] = (e * inv - shift) * gv[sl] + bv[sl]

                return 0

            lax.fori_loop(0, CHUNK, row_body, 0)

        start_in(0, 0)
        start_in(1, 1)

        def task_body(t, _):
            slot = lax.bitwise_and(t, 1)

            @pl.when(lax.bitwise_and(t, B - 1) == 0)
            def _():
                srow = s_base + lax.shift_right_logical(t, 2) * CHUNK
                pltpu.sync_copy(pos_hbm.at[pl.ds(srow, CHUNK)], pv)

            wait_chunk(si)

            @pl.when(t >= 2)
            def _():
                wait_chunk(so)

            compute_chunk(slot)
            pltpu.make_async_copy(
                ov.at[slot], out_hbm.at[pl.ds(task_row(t), CHUNK)], so
            ).start()

            @pl.when(t + 2 < T)
            def _():
                start_in(t + 2, slot)

            return 0

        lax.fori_loop(0, T, task_body, 0)
        wait_chunk(so)
        wait_chunk(so)

    return k(x2, pos_emb, ln_gamma, ln_beta).reshape(B, S, D)


# speedup vs baseline: 1.5755x; 1.0359x over previous
import functools
import jax
import jax.numpy as jnp
from jax import lax
from jax.experimental import pallas as pl
from jax.experimental.pallas import tpu as pltpu
from jax.experimental.pallas import tpu_sc as plsc

EPS = 1e-5
NC, NS, L = 2, 16, 16
NW = NC * NS
CHUNK = 16


def _rsqrt_nr(t):
    i = lax.bitcast_convert_type(t, jnp.int32)
    i = jnp.int32(0x5F3759DF) - lax.shift_right_arithmetic(i, 1)
    y = lax.bitcast_convert_type(i, jnp.float32)
    half = t * 0.5
    for _ in range(3):
        y = y * (1.5 - half * y * y)
    return y


def kernel(x, pos_emb, ln_gamma, ln_beta):
    B, S, D = x.shape
    vecs = D // L
    x2 = x.reshape(B * S, D)
    s_per_w = S // NW
    n_chunks = s_per_w // CHUNK
    T = n_chunks * B
    inv_d = 1.0 / D
    mesh = plsc.VectorSubcoreMesh(core_axis_name="c", subcore_axis_name="s")

    @functools.partial(
        pl.kernel,
        mesh=mesh,
        compiler_params=pltpu.CompilerParams(needs_layout_passes=False),
        out_type=jax.ShapeDtypeStruct((B * S, D), jnp.float32),
        scratch_types=[
            pltpu.VMEM((2, CHUNK, D), jnp.float32),
            pltpu.VMEM((2, CHUNK, D), jnp.float32),
            pltpu.VMEM((CHUNK, D), jnp.float32),
            pltpu.VMEM((D,), jnp.float32),
            pltpu.VMEM((D,), jnp.float32),
            pltpu.VMEM((CHUNK, L), jnp.float32),
            pltpu.SemaphoreType.DMA,
            pltpu.SemaphoreType.DMA,
        ],
    )
    def k(x_hbm, pos_hbm, g_hbm, b_hbm, out_hbm, xv, ov, pv, gv, bv, red, si, so):
        wid = lax.axis_index("s") * NC + lax.axis_index("c")
        s_base = wid * s_per_w
        pltpu.sync_copy(g_hbm, gv)
        pltpu.sync_copy(b_hbm, bv)
        iota = lax.iota(jnp.int32, L)

        def task_row(u):
            b = lax.bitwise_and(u, B - 1)
            c = lax.shift_right_logical(u, 2)
            return b * S + s_base + c * CHUNK

        def start_in(u, slot):
            pltpu.make_async_copy(
                x_hbm.at[pl.ds(task_row(u), CHUNK)], xv.at[slot], si
            ).start()

        def wait_chunk(sem):
            pltpu.make_async_copy(
                x_hbm.at[pl.ds(0, CHUNK)], xv.at[0], sem
            ).wait()

        dnums = lax.GatherDimensionNumbers(
            offset_dims=(), collapsed_slice_dims=(0,), start_index_map=(0,))

        def lanesum(v):
            for k_ in (1, 2, 4, 8):
                perm = lax.gather(
                    v, lax.bitwise_xor(iota, k_)[:, None], dnums,
                    slice_sizes=(1,),
                    mode=lax.GatherScatterMode.PROMISE_IN_BOUNDS)
                v = v + perm
            return v

        def compute_chunk(slot):
            def row_body(i, _):
                zero = jnp.zeros((L,), jnp.float32)

                @plsc.parallel_loop(0, vecs, unroll=8, carry=(zero, zero))
                def acc(j, c):
                    a1, a2 = c
                    sl = pl.ds(j * L, L)
                    e = xv[slot, i, sl] + pv[i, sl]
                    ov[slot, i, sl] = e
                    return a1 + e, a2 + e * e

                a1, a2 = acc
                mean = lanesum(a1) * inv_d
                msq = lanesum(a2) * inv_d
                inv = _rsqrt_nr(msq - mean * mean + EPS)
                shift = mean * inv

                @plsc.parallel_loop(0, vecs, unroll=8)
                def norm(j):
                    sl = pl.ds(j * L, L)
                    e = ov[slot, i, sl]
                    ov[slot, i, sl] = (e * inv - shift) * gv[sl] + bv[sl]

                return 0

            lax.fori_loop(0, CHUNK, row_body, 0)

        start_in(0, 0)
        start_in(1, 1)

        def task_body(t, _):
            slot = lax.bitwise_and(t, 1)

            @pl.when(lax.bitwise_and(t, B - 1) == 0)
            def _():
                srow = s_base + lax.shift_right_logical(t, 2) * CHUNK
                pltpu.sync_copy(pos_hbm.at[pl.ds(srow, CHUNK)], pv)

            wait_chunk(si)

            @pl.when(t >= 2)
            def _():
                wait_chunk(so)

            compute_chunk(slot)
            pltpu.make_async_copy(
                ov.at[slot], out_hbm.at[pl.ds(task_row(t), CHUNK)], so
            ).start()

            @pl.when(t + 2 < T)
            def _():
                start_in(t + 2, slot)

            return 0

        lax.fori_loop(0, T, task_body, 0)
        wait_chunk(so)
        wait_chunk(so)

    return k(x2, pos_emb, ln_gamma, ln_beta).reshape(B, S, D)


# speedup vs baseline: 1.5888x; 1.0085x over previous
import functools
import jax
import jax.numpy as jnp
from jax import lax
from jax.experimental import pallas as pl
from jax.experimental.pallas import tpu as pltpu
from jax.experimental.pallas import tpu_sc as plsc

EPS = 1e-5
NC, NS, L = 2, 16, 16
NW = NC * NS
CHUNK = 16


def _rsqrt_nr(t):
    i = lax.bitcast_convert_type(t, jnp.int32)
    i = jnp.int32(0x5F3759DF) - lax.shift_right_arithmetic(i, 1)
    y = lax.bitcast_convert_type(i, jnp.float32)
    half = t * 0.5
    for _ in range(3):
        y = y * (1.5 - half * y * y)
    return y


def kernel(x, pos_emb, ln_gamma, ln_beta):
    B, S, D = x.shape
    vecs = D // L
    x2 = x.reshape(B * S, D)
    s_per_w = S // NW
    n_chunks = s_per_w // CHUNK
    T = n_chunks * B
    inv_d = 1.0 / D
    mesh = plsc.VectorSubcoreMesh(core_axis_name="c", subcore_axis_name="s")

    @functools.partial(
        pl.kernel,
        mesh=mesh,
        compiler_params=pltpu.CompilerParams(needs_layout_passes=False),
        out_type=jax.ShapeDtypeStruct((B * S, D), jnp.float32),
        scratch_types=[
            pltpu.VMEM((2, CHUNK, D), jnp.float32),
            pltpu.VMEM((2, CHUNK, D), jnp.float32),
            pltpu.VMEM((CHUNK, D), jnp.float32),
            pltpu.VMEM((D,), jnp.float32),
            pltpu.VMEM((D,), jnp.float32),
            pltpu.VMEM((CHUNK, L), jnp.float32),
            pltpu.SemaphoreType.DMA,
            pltpu.SemaphoreType.DMA,
        ],
    )
    def k(x_hbm, pos_hbm, g_hbm, b_hbm, out_hbm, xv, ov, pv, gv, bv, red, si, so):
        wid = lax.axis_index("s") * NC + lax.axis_index("c")
        s_base = wid * s_per_w
        pltpu.sync_copy(g_hbm, gv)
        pltpu.sync_copy(b_hbm, bv)
        iota = lax.iota(jnp.int32, L)

        def task_row(u):
            b = lax.bitwise_and(u, B - 1)
            c = lax.shift_right_logical(u, 2)
            return b * S + s_base + c * CHUNK

        def start_in(u, slot):
            pltpu.make_async_copy(
                x_hbm.at[pl.ds(task_row(u), CHUNK)], xv.at[slot], si
            ).start()

        def wait_chunk(sem):
            pltpu.make_async_copy(
                x_hbm.at[pl.ds(0, CHUNK)], xv.at[0], sem
            ).wait()

        dnums = lax.GatherDimensionNumbers(
            offset_dims=(), collapsed_slice_dims=(0,), start_index_map=(0,))

        def lanesum(v):
            for k_ in (1, 2, 4, 8):
                perm = lax.gather(
                    v, lax.bitwise_xor(iota, k_)[:, None], dnums,
                    slice_sizes=(1,),
                    mode=lax.GatherScatterMode.PROMISE_IN_BOUNDS)
                v = v + perm
            return v

        def compute_chunk(slot):
            @plsc.parallel_loop(0, CHUNK, unroll=2)
            def row_body(i):
                zero = jnp.zeros((L,), jnp.float32)

                @plsc.parallel_loop(0, vecs, unroll=8, carry=(zero, zero))
                def acc(j, c):
                    a1, a2 = c
                    sl = pl.ds(j * L, L)
                    e = xv[slot, i, sl] + pv[i, sl]
                    ov[slot, i, sl] = e
                    return a1 + e, a2 + e * e

                a1, a2 = acc
                mean = lanesum(a1) * inv_d
                msq = lanesum(a2) * inv_d
                inv = _rsqrt_nr(msq - mean * mean + EPS)
                shift = mean * inv

                @plsc.parallel_loop(0, vecs, unroll=8)
                def norm(j):
                    sl = pl.ds(j * L, L)
                    e = ov[slot, i, sl]
                    ov[slot, i, sl] = (e * inv - shift) * gv[sl] + bv[sl]

        start_in(0, 0)
        start_in(1, 1)

        def task_body(t, _):
            slot = lax.bitwise_and(t, 1)

            @pl.when(lax.bitwise_and(t, B - 1) == 0)
            def _():
                srow = s_base + lax.shift_right_logical(t, 2) * CHUNK
                pltpu.sync_copy(pos_hbm.at[pl.ds(srow, CHUNK)], pv)

            wait_chunk(si)

            @pl.when(t >= 2)
            def _():
                wait_chunk(so)

            compute_chunk(slot)
            pltpu.make_async_copy(
                ov.at[slot], out_hbm.at[pl.ds(task_row(t), CHUNK)], so
            ).start()

            @pl.when(t + 2 < T)
            def _():
                start_in(t + 2, slot)

            return 0

        lax.fori_loop(0, T, task_body, 0)
        wait_chunk(so)
        wait_chunk(so)

    return k(x2, pos_emb, ln_gamma, ln_beta).reshape(B, S, D)


# speedup vs baseline: 4.5584x; 2.8690x over previous
import jax
import jax.numpy as jnp
from jax.experimental import pallas as pl
from jax.experimental.pallas import tpu as pltpu

EPS = 1e-5
ROWS = 2048


def _ln_kernel(x_ref, pos_ref, gamma_ref, beta_ref, out_ref):
    e = x_ref[0] + pos_ref[...]
    mean = jnp.mean(e, axis=-1, keepdims=True)
    c = e - mean
    var = jnp.mean(c * c, axis=-1, keepdims=True)
    inv = jax.lax.rsqrt(var + EPS)
    out_ref[0] = c * inv * gamma_ref[...] + beta_ref[...]


def kernel(x, pos_emb, ln_gamma, ln_beta):
    B, S, D = x.shape
    gamma2 = ln_gamma.reshape(1, D)
    beta2 = ln_beta.reshape(1, D)
    grid = (S // ROWS, B)
    return pl.pallas_call(
        _ln_kernel,
        grid=grid,
        in_specs=[
            pl.BlockSpec((1, ROWS, D), lambda j, b: (b, j, 0)),
            pl.BlockSpec((ROWS, D), lambda j, b: (j, 0)),
            pl.BlockSpec((1, D), lambda j, b: (0, 0)),
            pl.BlockSpec((1, D), lambda j, b: (0, 0)),
        ],
        out_specs=pl.BlockSpec((1, ROWS, D), lambda j, b: (b, j, 0)),
        out_shape=jax.ShapeDtypeStruct((B, S, D), x.dtype),
        compiler_params=pltpu.CompilerParams(
            dimension_semantics=("parallel", "arbitrary"),
        ),
    )(x, pos_emb[:S], gamma2, beta2)
